# Initial kernel scaffold; baseline (speedup 1.0000x reference)
import jax
import jax.numpy as jnp
from jax.experimental import pallas as pl


def kernel(x, embedding_weight):
    raise NotImplementedError("write your pallas kernel here")


# speedup vs baseline: 2.0409x; 2.0409x over previous
import jax
import jax.numpy as jnp
from jax.experimental import pallas as pl

_TM = 256


def _vq_body(x2_ref, e2_ref, x_ref, e_ref, dist_ref, idx_ref):
    mm = jax.lax.dot_general(
        x_ref[...], e_ref[...],
        dimension_numbers=(((1,), (1,)), ((), ())),
        preferred_element_type=jnp.float32)
    d = (x2_ref[...] + e2_ref[...]) - 2.0 * mm
    dist_ref[...] = d
    k = d.shape[1]
    col = jax.lax.broadcasted_iota(jnp.int32, d.shape, 1)
    dmin = jnp.min(d, axis=1, keepdims=True)
    idx_ref[...] = jnp.min(jnp.where(d == dmin, col, k), axis=1)


def kernel(x, embedding_weight):
    B, C, H, W = x.shape
    K, D = embedding_weight.shape
    M = B * H * W
    x_flat = jnp.transpose(x.reshape(B, C, H * W), (0, 2, 1))
    x2 = jnp.sum(x_flat ** 2, axis=2, keepdims=True)
    e2 = jnp.sum(embedding_weight ** 2, axis=1)
    xm = x_flat.reshape(M, D)
    x2m = x2.reshape(M, 1)
    e2m = e2.reshape(1, K)
    dist, idx = pl.pallas_call(
        _vq_body,
        grid=(M // _TM,),
        in_specs=[
            pl.BlockSpec((_TM, 1), lambda i: (i, 0)),
            pl.BlockSpec((1, K), lambda i: (0, 0)),
            pl.BlockSpec((_TM, D), lambda i: (i, 0)),
            pl.BlockSpec((K, D), lambda i: (0, 0)),
        ],
        out_specs=[
            pl.BlockSpec((_TM, K), lambda i: (i, 0)),
            pl.BlockSpec((_TM,), lambda i: (i,)),
        ],
        out_shape=[
            jax.ShapeDtypeStruct((M, K), jnp.float32),
            jax.ShapeDtypeStruct((M,), jnp.int32),
        ],
    )(x2m, e2m, xm, embedding_weight)
    return (idx.reshape(B, H * W), dist.reshape(B, H * W, K))
